# baseline (speedup 1.0000x reference)
import functools
import math

import jax
import jax.numpy as jnp
from jax import lax
from jax.experimental import pallas as pl
from jax.experimental.pallas import tpu as pltpu
from jax.experimental.pallas import tpu_sc as plsc

_V = 100000
_PADDING_IDX = 1
_SMOOTHING = 0.1
_CONF = 1.0 - _SMOOTHING
_EPS = _SMOOTHING / _V
_C1 = (_V - 1) * _EPS * math.log(_EPS) + _CONF * math.log(_CONF)

_N = 512
_VB = 2048
_NB = -(-_V // _VB)

_NC, _NS, _L = 2, 16, 16
_NW = _NC * _NS
_RPW = _N // _NW


def _stats_body(x_ref, logz_ref, sx_ref, m_sc, s_sc, sx_sc):
    j = pl.program_id(0)

    @pl.when(j == 0)
    def _init():
        m_sc[...] = jnp.full((_N, 1), -jnp.inf, jnp.float32)
        s_sc[...] = jnp.zeros((_N, 1), jnp.float32)
        sx_sc[...] = jnp.zeros((_N, 1), jnp.float32)

    x = x_ref[...]

    @pl.when(j < _NB - 1)
    def _full():
        m0 = m_sc[...]
        mn = jnp.maximum(m0, jnp.max(x, axis=1, keepdims=True))
        s_sc[...] = (s_sc[...] * jnp.exp(m0 - mn)
                     + jnp.sum(jnp.exp(x - mn), axis=1, keepdims=True))
        m_sc[...] = mn
        sx_sc[...] += jnp.sum(x, axis=1, keepdims=True)

    @pl.when(j == _NB - 1)
    def _tail():
        lane = jax.lax.broadcasted_iota(jnp.int32, (_N, _VB), 1)
        valid = lane < (_V - j * _VB)
        xm = jnp.where(valid, x, -jnp.inf)
        m0 = m_sc[...]
        mn = jnp.maximum(m0, jnp.max(xm, axis=1, keepdims=True))
        s = (s_sc[...] * jnp.exp(m0 - mn)
             + jnp.sum(jnp.exp(xm - mn), axis=1, keepdims=True))
        sx = sx_sc[...] + jnp.sum(jnp.where(valid, x, 0.0),
                                  axis=1, keepdims=True)
        logz_ref[...] = mn + jnp.log(s)
        sx_ref[...] = sx


_sc_mesh = plsc.VectorSubcoreMesh(core_axis_name="c", subcore_axis_name="s")


@functools.partial(
    pl.kernel,
    out_type=jax.ShapeDtypeStruct((_N, 128), jnp.float32),
    mesh=_sc_mesh,
    scratch_types=[
        pltpu.VMEM((_RPW,), jnp.int32),
        pltpu.VMEM((_RPW, 128), jnp.float32),
        pltpu.VMEM((_RPW,), jnp.int32),
        pltpu.SemaphoreType.DMA,
    ],
)
def _sc_gather(pred128_hbm, t_hbm, out_hbm, idx_v, rows_v, t_v, sem):
    wid = lax.axis_index("s") * _NC + lax.axis_index("c")
    base = wid * _RPW
    pltpu.sync_copy(t_hbm.at[pl.ds(base, _RPW)], t_v)
    flat = (lax.iota(jnp.int32, _L) + base) * _V + t_v[...]
    idx_v[...] = lax.shift_right_logical(flat, 7)
    pltpu.async_copy(pred128_hbm.at[idx_v], rows_v, sem).wait()
    pltpu.sync_copy(rows_v, out_hbm.at[pl.ds(base, _RPW)])


def _finalize_body(logz_ref, sx_ref, rows_ref, t_ref, dl_ref, out_ref):
    logz = logz_ref[...]
    rowid = jax.lax.broadcasted_iota(jnp.int32, (_N, 1), 0)
    tgt_lane = jnp.bitwise_and(rowid * _V + t_ref[...], 127)
    lane = jax.lax.broadcasted_iota(jnp.int32, (_N, 128), 1)
    xt = jnp.sum(jnp.where(lane == tgt_lane, rows_ref[...], 0.0),
                 axis=1, keepdims=True)
    denom = jnp.sum(dl_ref[...], axis=0, keepdims=True)
    row_kl = (jnp.float32(_C1)
              - jnp.float32(_EPS) * (sx_ref[...]
                                     - jnp.float32(_V) * logz)
              - jnp.float32(_CONF - _EPS) * (xt - logz))
    row_kl = jnp.where(t_ref[...] == _PADDING_IDX, 0.0, row_kl)
    out_ref[...] = jnp.sum(row_kl, axis=0, keepdims=True) / denom


def kernel(pred, targets, decode_lengths):
    x = pred.reshape(_N, _V)
    t1 = targets.reshape(_N).astype(jnp.int32)
    t2 = t1.reshape(_N, 1)
    dl = decode_lengths.reshape(-1, 1).astype(jnp.float32)

    logz, sx = pl.pallas_call(
        _stats_body,
        grid=(_NB,),
        in_specs=[
            pl.BlockSpec((_N, _VB), lambda j: (0, j)),
        ],
        out_specs=[
            pl.BlockSpec((_N, 1), lambda j: (0, 0)),
            pl.BlockSpec((_N, 1), lambda j: (0, 0)),
        ],
        out_shape=[
            jax.ShapeDtypeStruct((_N, 1), jnp.float32),
            jax.ShapeDtypeStruct((_N, 1), jnp.float32),
        ],
        scratch_shapes=[
            pltpu.VMEM((_N, 1), jnp.float32),
            pltpu.VMEM((_N, 1), jnp.float32),
            pltpu.VMEM((_N, 1), jnp.float32),
        ],
        compiler_params=pltpu.CompilerParams(
            dimension_semantics=("arbitrary",),
        ),
    )(x)

    rows = _sc_gather(pred.reshape(_N * _V // 128, 128), t1)

    out = pl.pallas_call(
        _finalize_body,
        out_shape=jax.ShapeDtypeStruct((1, 1), jnp.float32),
    )(logz, sx, rows, t2, dl)
    return out.reshape(())


# speedup vs baseline: 3.9731x; 3.9731x over previous
import math

import jax
import jax.numpy as jnp
from jax.experimental import pallas as pl
from jax.experimental.pallas import tpu as pltpu

_V = 100000
_PADDING_IDX = 1
_SMOOTHING = 0.1
_CONF = 1.0 - _SMOOTHING
_EPS = _SMOOTHING / _V
_C1 = (_V - 1) * _EPS * math.log(_EPS) + _CONF * math.log(_CONF)

_N = 512
_VB = 4096
_NB = -(-_V // _VB)


def _stats_body(x_ref, t_ref, dl_ref, out_ref, m_sc, s_sc, sx_sc, xt_sc):
    j = pl.program_id(0)

    @pl.when(j == 0)
    def _init():
        m_sc[...] = jnp.full((_N, 1), -jnp.inf, jnp.float32)
        s_sc[...] = jnp.zeros((_N, 1), jnp.float32)
        sx_sc[...] = jnp.zeros((_N, 1), jnp.float32)
        xt_sc[...] = jnp.zeros((_N, 1), jnp.float32)

    x = x_ref[...]
    t_loc = t_ref[...] - j * _VB
    lane = jax.lax.broadcasted_iota(jnp.int32, (_N, _VB), 1)

    @pl.when(j < _NB - 1)
    def _full():
        m0 = m_sc[...]
        mn = jnp.maximum(m0, jnp.max(x, axis=1, keepdims=True))
        s_sc[...] = (s_sc[...] * jnp.exp(m0 - mn)
                     + jnp.sum(jnp.exp(x - mn), axis=1, keepdims=True))
        m_sc[...] = mn
        sx_sc[...] += jnp.sum(x, axis=1, keepdims=True)
        xt_sc[...] += jnp.sum(jnp.where(lane == t_loc, x, 0.0),
                              axis=1, keepdims=True)

    @pl.when(j == _NB - 1)
    def _tail_and_finalize():
        valid = lane < (_V - j * _VB)
        xm = jnp.where(valid, x, -jnp.inf)
        x0 = jnp.where(valid, x, 0.0)
        m0 = m_sc[...]
        mn = jnp.maximum(m0, jnp.max(xm, axis=1, keepdims=True))
        s = (s_sc[...] * jnp.exp(m0 - mn)
             + jnp.sum(jnp.exp(xm - mn), axis=1, keepdims=True))
        sx = sx_sc[...] + jnp.sum(x0, axis=1, keepdims=True)
        xt = xt_sc[...] + jnp.sum(jnp.where(lane == t_loc, x0, 0.0),
                                  axis=1, keepdims=True)
        logz = mn + jnp.log(s)
        row_kl = (jnp.float32(_C1)
                  - jnp.float32(_EPS) * (sx - jnp.float32(_V) * logz)
                  - jnp.float32(_CONF - _EPS) * (xt - logz))
        row_kl = jnp.where(t_ref[...] == _PADDING_IDX, 0.0, row_kl)
        denom = jnp.sum(dl_ref[...], axis=0, keepdims=True)
        out_ref[...] = jnp.sum(row_kl, axis=0, keepdims=True) / denom


def kernel(pred, targets, decode_lengths):
    x = pred.reshape(_N, _V)
    t = targets.reshape(_N, 1).astype(jnp.int32)
    dl = decode_lengths.reshape(-1, 1).astype(jnp.float32)

    out = pl.pallas_call(
        _stats_body,
        grid=(_NB,),
        in_specs=[
            pl.BlockSpec((_N, _VB), lambda j: (0, j)),
            pl.BlockSpec((_N, 1), lambda j: (0, 0)),
            pl.BlockSpec((dl.shape[0], 1), lambda j: (0, 0)),
        ],
        out_specs=pl.BlockSpec((1, 1), lambda j: (0, 0)),
        out_shape=jax.ShapeDtypeStruct((1, 1), jnp.float32),
        scratch_shapes=[
            pltpu.VMEM((_N, 1), jnp.float32),
            pltpu.VMEM((_N, 1), jnp.float32),
            pltpu.VMEM((_N, 1), jnp.float32),
            pltpu.VMEM((_N, 1), jnp.float32),
        ],
        compiler_params=pltpu.CompilerParams(
            dimension_semantics=("arbitrary",),
        ),
    )(x, t, dl)
    return out.reshape(())
